# Initial kernel scaffold; baseline (speedup 1.0000x reference)
import jax
import jax.numpy as jnp
from jax.experimental import pallas as pl


def kernel(x, alpha, edge_index, batch, W_pre, b_pre, Wl1, bl1, Wr1, Wl2, bl2, Wr2, Wl3, bl3, Wr3, W_hh1, b_hh1, W_hh2, b_hh2, W_oh, b_oh, W_h1, b_h1):
    raise NotImplementedError("write your pallas kernel here")


# speedup vs baseline: 3.0986x; 3.0986x over previous
import functools

import jax
import jax.numpy as jnp
from jax import lax
from jax.experimental import pallas as pl
from jax.experimental.pallas import tpu as pltpu
from jax.experimental.pallas import tpu_sc as plsc

N = 10000
E = 160000
NCHUNK = 4
CW = 128
NT = 16
EP = E // NT
KB = 128
NB = -(-EP // KB)
EPP = NB * KB
NPAD = 10240
RPT = NPAD // NT


def _seg_sum_body(h, srcp, dstp, zhbm, out, idx_cur, dst_cur, gbuf, acc, sem):
    s = lax.axis_index("s")
    c = lax.axis_index("c")

    for cabs in range(NCHUNK):
        own = c == cabs // (NCHUNK // 2)

        @pl.when(own)
        def _zero(cabs=cabs):
            for j in range(RPT // KB):
                pltpu.sync_copy(zhbm, acc.at[pl.ds(s * RPT + j * KB, KB)])

        plsc.subcore_barrier()

        @pl.when(own)
        def _scan(cabs=cabs):
            def batch_body(b, carry):
                pltpu.sync_copy(srcp.at[s, pl.ds(b * KB, KB)], idx_cur)
                pltpu.sync_copy(dstp.at[s, pl.ds(b * KB, KB)], dst_cur)
                pltpu.async_copy(h.at[idx_cur, pl.ds(cabs * CW, CW)],
                                 gbuf, sem).wait()
                pltpu.sync_copy(gbuf, acc.at[dst_cur], add=True)
                return carry

            lax.fori_loop(0, NB, batch_body, 0)

        plsc.subcore_barrier()

        @pl.when(own)
        def _flush(cabs=cabs):
            for j in range(RPT // KB):
                r0 = s * RPT + j * KB
                pltpu.sync_copy(acc.at[pl.ds(r0, KB)],
                                out.at[pl.ds(r0, KB), pl.ds(cabs * CW, CW)])

        plsc.subcore_barrier()


def _seg_scratch():
    return [
        pltpu.VMEM((KB,), jnp.int32),
        pltpu.VMEM((KB,), jnp.int32),
        pltpu.VMEM((KB, CW), jnp.float32),
        pltpu.VMEM_SHARED((NPAD, CW), jnp.float32),
        pltpu.SemaphoreType.DMA,
    ]


def _seg_sum(h, srcp, dstp):
    zhbm = jnp.zeros((KB, CW), jnp.float32)
    mesh = plsc.VectorSubcoreMesh(core_axis_name="c", subcore_axis_name="s")
    return pl.kernel(
        _seg_sum_body,
        out_type=jax.ShapeDtypeStruct((NPAD, 512), jnp.float32),
        mesh=mesh,
        scratch_types=_seg_scratch(),
    )(h, srcp, dstp, zhbm)


def _cnt_body(dstp, onesb, zhbm, out, dst_cur, onesv, acc, sem):
    s = lax.axis_index("s")
    c = lax.axis_index("c")

    @pl.when(c == 0)
    def _zero():
        for j in range(RPT // KB):
            pltpu.sync_copy(zhbm, acc.at[pl.ds(s * RPT + j * KB, KB)])
        pltpu.sync_copy(onesb, onesv)

    plsc.subcore_barrier()

    @pl.when(c == 0)
    def _scan():
        def batch_body(b, carry):
            pltpu.sync_copy(dstp.at[s, pl.ds(b * KB, KB)], dst_cur)
            pltpu.sync_copy(onesv, acc.at[dst_cur], add=True)
            return carry

        lax.fori_loop(0, NB, batch_body, 0)

    plsc.subcore_barrier()

    @pl.when(c == 0)
    def _flush():
        for j in range(RPT // KB):
            r0 = s * RPT + j * KB
            pltpu.sync_copy(acc.at[pl.ds(r0, KB)], out.at[pl.ds(r0, KB)])


def _sc_counts(dstp):
    onesb = jnp.ones((KB, CW), jnp.float32)
    zhbm = jnp.zeros((KB, CW), jnp.float32)
    mesh = plsc.VectorSubcoreMesh(core_axis_name="c", subcore_axis_name="s")
    return pl.kernel(
        _cnt_body,
        out_type=jax.ShapeDtypeStruct((NPAD, CW), jnp.float32),
        mesh=mesh,
        scratch_types=[
            pltpu.VMEM((KB,), jnp.int32),
            pltpu.VMEM((KB, CW), jnp.float32),
            pltpu.VMEM_SHARED((NPAD, CW), jnp.float32),
            pltpu.SemaphoreType.DMA,
        ],
    )(dstp, onesb, zhbm)


_R = 1000


def _tc_pre(x, Wp, bp):
    def body(x_ref, w_ref, b_ref, o_ref):
        o_ref[...] = jnp.dot(x_ref[...], w_ref[...],
                             preferred_element_type=jnp.float32) + b_ref[...]

    return pl.pallas_call(
        body,
        grid=(N // _R,),
        in_specs=[pl.BlockSpec((_R, 256), lambda i: (i, 0)),
                  pl.BlockSpec((256, 512), lambda i: (0, 0)),
                  pl.BlockSpec((1, 512), lambda i: (0, 0))],
        out_specs=pl.BlockSpec((_R, 512), lambda i: (i, 0)),
        out_shape=jax.ShapeDtypeStruct((N, 512), jnp.float32),
    )(x, Wp, bp.reshape(1, 512))


def _tc_layer(cnt16, agg, h, Wl, bl, Wr, Whh, bhh):
    HO = Wl.shape[1]

    def body(c_ref, a_ref, h_ref, wl_ref, bl_ref, wr_ref, wh_ref, bh_ref,
             o_ref):
        inv = 1.0 / jnp.maximum(c_ref[...][:, 0:1], 1.0)
        mean = a_ref[...] * inv
        t = (jnp.dot(mean, wl_ref[...], preferred_element_type=jnp.float32)
             + bl_ref[...]
             + jnp.dot(h_ref[...], wr_ref[...],
                       preferred_element_type=jnp.float32))
        t = jnp.maximum(t, 0.0)
        u = jnp.dot(t, wh_ref[...], preferred_element_type=jnp.float32) \
            + bh_ref[...]
        o_ref[...] = jnp.where(u >= 0, u, 0.01 * u)

    return pl.pallas_call(
        body,
        grid=(N // _R,),
        in_specs=[pl.BlockSpec((_R, 128), lambda i: (i, 0)),
                  pl.BlockSpec((_R, 512), lambda i: (i, 0)),
                  pl.BlockSpec((_R, 512), lambda i: (i, 0)),
                  pl.BlockSpec((512, HO), lambda i: (0, 0)),
                  pl.BlockSpec((1, HO), lambda i: (0, 0)),
                  pl.BlockSpec((512, HO), lambda i: (0, 0)),
                  pl.BlockSpec((HO, 512), lambda i: (0, 0)),
                  pl.BlockSpec((1, 512), lambda i: (0, 0))],
        out_specs=pl.BlockSpec((_R, 512), lambda i: (i, 0)),
        out_shape=jax.ShapeDtypeStruct((N, 512), jnp.float32),
    )(cnt16, agg, h, Wl, bl.reshape(1, HO), Wr, Whh, bhh.reshape(1, 512))


def _tc_final(cnt16, agg, h, Wl3, bl3, Wr3, Woh, boh, batch2, alpha2, w1a,
              w1b, b11, noise, mask):
    def body(c_ref, a_ref, h_ref, wl_ref, bl_ref, wr_ref, wo_ref, bo_ref,
             bt_ref, al_ref, wa_ref, wb_ref, bb_ref, nz_ref, mk_ref, o_ref):
        inv = 1.0 / jnp.maximum(c_ref[...][:, 0:1], 1.0)
        mean = a_ref[...] * inv
        t = (jnp.dot(mean, wl_ref[...], preferred_element_type=jnp.float32)
             + bl_ref[...]
             + jnp.dot(h_ref[...], wr_ref[...],
                       preferred_element_type=jnp.float32))
        t = jnp.maximum(t, 0.0)
        hf = jnp.dot(t, wo_ref[...], preferred_element_type=jnp.float32) \
            + bo_ref[...]
        hf = jnp.where(hf >= 0, hf, 0.01 * hf)
        bi = lax.broadcasted_iota(jnp.int32, (_R, 64), 1)
        oh = (bt_ref[...] == bi).astype(jnp.float32)
        an = jnp.dot(oh, al_ref[...], preferred_element_type=jnp.float32)
        z = (jnp.dot(hf, wa_ref[...], preferred_element_type=jnp.float32)
             + an * wb_ref[...] + bb_ref[...] + nz_ref[...])
        o_ref[...] = 10.0 * jax.nn.sigmoid(z) * mk_ref[...]

    return pl.pallas_call(
        body,
        grid=(N // _R,),
        in_specs=[pl.BlockSpec((_R, 128), lambda i: (i, 0)),
                  pl.BlockSpec((_R, 512), lambda i: (i, 0)),
                  pl.BlockSpec((_R, 512), lambda i: (i, 0)),
                  pl.BlockSpec((512, 256), lambda i: (0, 0)),
                  pl.BlockSpec((1, 256), lambda i: (0, 0)),
                  pl.BlockSpec((512, 256), lambda i: (0, 0)),
                  pl.BlockSpec((256, 512), lambda i: (0, 0)),
                  pl.BlockSpec((1, 512), lambda i: (0, 0)),
                  pl.BlockSpec((_R, 1), lambda i: (i, 0)),
                  pl.BlockSpec((64, 1), lambda i: (0, 0)),
                  pl.BlockSpec((512, 1), lambda i: (0, 0)),
                  pl.BlockSpec((1, 1), lambda i: (0, 0)),
                  pl.BlockSpec((1, 1), lambda i: (0, 0)),
                  pl.BlockSpec((_R, 1), lambda i: (i, 0)),
                  pl.BlockSpec((_R, 1), lambda i: (i, 0))],
        out_specs=pl.BlockSpec((_R, 1), lambda i: (i, 0)),
        out_shape=jax.ShapeDtypeStruct((N, 1), jnp.float32),
    )(cnt16, agg, h, Wl3, bl3.reshape(1, 256), Wr3, Woh, boh.reshape(1, 512),
      batch2, alpha2, w1a, w1b, b11, noise, mask)


def kernel(x, alpha, edge_index, batch, W_pre, b_pre, Wl1, bl1, Wr1, Wl2, bl2,
           Wr2, Wl3, bl3, Wr3, W_hh1, b_hh1, W_hh2, b_hh2, W_oh, b_oh, W_h1,
           b_h1):
    src, dst = edge_index[0], edge_index[1]
    fixed_feature = x[:, :10]
    mask = x[:, 9:10]

    u = jax.random.uniform(jax.random.key(12345), (N, 1), minval=1e-10,
                           maxval=1.0 - 1e-10, dtype=jnp.float32)
    noise = jnp.log(u) - jnp.log(1.0 - u)

    npad_e = EPP - EP
    padsrc = jnp.arange(npad_e, dtype=jnp.int32).reshape(1, npad_e)
    paddst = (N + jnp.arange(npad_e, dtype=jnp.int32)).reshape(1, npad_e)
    srcp = jnp.concatenate(
        [src.reshape(NT, EP), jnp.broadcast_to(padsrc, (NT, npad_e))], axis=1)
    dstp = jnp.concatenate(
        [dst.reshape(NT, EP), jnp.broadcast_to(paddst, (NT, npad_e))], axis=1)

    h0 = _tc_pre(x, W_pre, b_pre)
    cnt128 = _sc_counts(dstp)
    agg1 = _seg_sum(h0, srcp, dstp)
    h1 = _tc_layer(cnt128, agg1, h0, Wl1, bl1, Wr1, W_hh1, b_hh1)
    agg2 = _seg_sum(h1, srcp, dstp)
    h2 = _tc_layer(cnt128, agg2, h1, Wl2, bl2, Wr2, W_hh2, b_hh2)
    agg3 = _seg_sum(h2, srcp, dstp)
    out = _tc_final(cnt128, agg3, h2, Wl3, bl3, Wr3, W_oh, b_oh,
                    batch.reshape(N, 1), alpha.reshape(64, 1),
                    W_h1[:512], W_h1[512:513], b_h1.reshape(1, 1),
                    noise, mask)
    return (out, fixed_feature)


# speedup vs baseline: 3.9131x; 1.2629x over previous
import functools

import jax
import jax.numpy as jnp
from jax import lax
from jax.experimental import pallas as pl
from jax.experimental.pallas import tpu as pltpu
from jax.experimental.pallas import tpu_sc as plsc

N = 10000
E = 160000
NCHUNK = 4
CW = 128
NT = 16
EP = E // NT
KB = 128
NB = 80
EPP = NB * KB
NPAD = 10240
RPT = NPAD // NT


def _seg_sum_body(h, srcp, dstp, zhbm, out, idx_a, idx_b, dst_a, dst_b,
                  gbuf_a, gbuf_b, acc, sem_a, sem_b):
    s = lax.axis_index("s")
    c = lax.axis_index("c")

    for cabs in range(NCHUNK):
        own = c == cabs // (NCHUNK // 2)
        cs = cabs * CW

        @pl.when(own)
        def _zero(cabs=cabs):
            for j in range(RPT // KB):
                pltpu.sync_copy(zhbm, acc.at[pl.ds(s * RPT + j * KB, KB)])

        plsc.subcore_barrier()

        @pl.when(own)
        def _scan(cabs=cabs, cs=cs):
            pltpu.sync_copy(srcp.at[s, pl.ds(0, KB)], idx_a)
            pltpu.sync_copy(dstp.at[s, pl.ds(0, KB)], dst_a)
            pltpu.async_copy(h.at[idx_a, pl.ds(cs, CW)], gbuf_a, sem_a)

            def pair_body(k, carry):
                b1 = 2 * k + 1
                pltpu.sync_copy(srcp.at[s, pl.ds(b1 * KB, KB)], idx_b)
                pltpu.sync_copy(dstp.at[s, pl.ds(b1 * KB, KB)], dst_b)
                pltpu.async_copy(h.at[idx_b, pl.ds(cs, CW)], gbuf_b, sem_b)
                pltpu.make_async_copy(h.at[idx_a, pl.ds(cs, CW)], gbuf_a,
                                      sem_a).wait()
                pltpu.sync_copy(gbuf_a, acc.at[dst_a], add=True)

                @pl.when(k < NB // 2 - 1)
                def _prefetch():
                    b2 = 2 * k + 2
                    pltpu.sync_copy(srcp.at[s, pl.ds(b2 * KB, KB)], idx_a)
                    pltpu.sync_copy(dstp.at[s, pl.ds(b2 * KB, KB)], dst_a)
                    pltpu.async_copy(h.at[idx_a, pl.ds(cs, CW)], gbuf_a,
                                     sem_a)

                pltpu.make_async_copy(h.at[idx_b, pl.ds(cs, CW)], gbuf_b,
                                      sem_b).wait()
                pltpu.sync_copy(gbuf_b, acc.at[dst_b], add=True)
                return carry

            lax.fori_loop(0, NB // 2, pair_body, 0)

        plsc.subcore_barrier()

        @pl.when(own)
        def _flush(cabs=cabs, cs=cs):
            for j in range(RPT // KB):
                r0 = s * RPT + j * KB
                pltpu.sync_copy(acc.at[pl.ds(r0, KB)],
                                out.at[pl.ds(r0, KB), pl.ds(cs, CW)])

        plsc.subcore_barrier()


def _seg_sum(h, srcp, dstp):
    zhbm = jnp.zeros((KB, CW), jnp.float32)
    mesh = plsc.VectorSubcoreMesh(core_axis_name="c", subcore_axis_name="s")
    return pl.kernel(
        _seg_sum_body,
        out_type=jax.ShapeDtypeStruct((NPAD, 512), jnp.float32),
        mesh=mesh,
        scratch_types=[
            pltpu.VMEM((KB,), jnp.int32),
            pltpu.VMEM((KB,), jnp.int32),
            pltpu.VMEM((KB,), jnp.int32),
            pltpu.VMEM((KB,), jnp.int32),
            pltpu.VMEM((KB, CW), jnp.float32),
            pltpu.VMEM((KB, CW), jnp.float32),
            pltpu.VMEM_SHARED((NPAD, CW), jnp.float32),
            pltpu.SemaphoreType.DMA,
            pltpu.SemaphoreType.DMA,
        ],
    )(h, srcp, dstp, zhbm)


def _cnt_body(dstp, onesb, zhbm, out, dst_cur, onesv, acc, sem):
    s = lax.axis_index("s")
    c = lax.axis_index("c")

    @pl.when(c == 0)
    def _zero():
        for j in range(RPT // KB):
            pltpu.sync_copy(zhbm, acc.at[pl.ds(s * RPT + j * KB, KB)])
        pltpu.sync_copy(onesb, onesv)

    plsc.subcore_barrier()

    @pl.when(c == 0)
    def _scan():
        def batch_body(b, carry):
            pltpu.sync_copy(dstp.at[s, pl.ds(b * KB, KB)], dst_cur)
            pltpu.sync_copy(onesv, acc.at[dst_cur], add=True)
            return carry

        lax.fori_loop(0, NB, batch_body, 0)

    plsc.subcore_barrier()

    @pl.when(c == 0)
    def _flush():
        for j in range(RPT // KB):
            r0 = s * RPT + j * KB
            pltpu.sync_copy(acc.at[pl.ds(r0, KB)], out.at[pl.ds(r0, KB)])


def _sc_counts(dstp):
    onesb = jnp.ones((KB, CW), jnp.float32)
    zhbm = jnp.zeros((KB, CW), jnp.float32)
    mesh = plsc.VectorSubcoreMesh(core_axis_name="c", subcore_axis_name="s")
    return pl.kernel(
        _cnt_body,
        out_type=jax.ShapeDtypeStruct((NPAD, CW), jnp.float32),
        mesh=mesh,
        scratch_types=[
            pltpu.VMEM((KB,), jnp.int32),
            pltpu.VMEM((KB, CW), jnp.float32),
            pltpu.VMEM_SHARED((NPAD, CW), jnp.float32),
            pltpu.SemaphoreType.DMA,
        ],
    )(dstp, onesb, zhbm)


_R = 1000


def _tc_pre(x, Wp, bp):
    def body(x_ref, w_ref, b_ref, o_ref):
        o_ref[...] = jnp.dot(x_ref[...], w_ref[...],
                             preferred_element_type=jnp.float32) + b_ref[...]

    return pl.pallas_call(
        body,
        grid=(N // _R,),
        in_specs=[pl.BlockSpec((_R, 256), lambda i: (i, 0)),
                  pl.BlockSpec((256, 512), lambda i: (0, 0)),
                  pl.BlockSpec((1, 512), lambda i: (0, 0))],
        out_specs=pl.BlockSpec((_R, 512), lambda i: (i, 0)),
        out_shape=jax.ShapeDtypeStruct((N, 512), jnp.float32),
    )(x, Wp, bp.reshape(1, 512))


def _tc_layer(cnt16, agg, h, Wl, bl, Wr, Whh, bhh):
    HO = Wl.shape[1]

    def body(c_ref, a_ref, h_ref, wl_ref, bl_ref, wr_ref, wh_ref, bh_ref,
             o_ref):
        inv = 1.0 / jnp.maximum(c_ref[...][:, 0:1], 1.0)
        mean = a_ref[...] * inv
        t = (jnp.dot(mean, wl_ref[...], preferred_element_type=jnp.float32)
             + bl_ref[...]
             + jnp.dot(h_ref[...], wr_ref[...],
                       preferred_element_type=jnp.float32))
        t = jnp.maximum(t, 0.0)
        u = jnp.dot(t, wh_ref[...], preferred_element_type=jnp.float32) \
            + bh_ref[...]
        o_ref[...] = jnp.where(u >= 0, u, 0.01 * u)

    return pl.pallas_call(
        body,
        grid=(N // _R,),
        in_specs=[pl.BlockSpec((_R, 128), lambda i: (i, 0)),
                  pl.BlockSpec((_R, 512), lambda i: (i, 0)),
                  pl.BlockSpec((_R, 512), lambda i: (i, 0)),
                  pl.BlockSpec((512, HO), lambda i: (0, 0)),
                  pl.BlockSpec((1, HO), lambda i: (0, 0)),
                  pl.BlockSpec((512, HO), lambda i: (0, 0)),
                  pl.BlockSpec((HO, 512), lambda i: (0, 0)),
                  pl.BlockSpec((1, 512), lambda i: (0, 0))],
        out_specs=pl.BlockSpec((_R, 512), lambda i: (i, 0)),
        out_shape=jax.ShapeDtypeStruct((N, 512), jnp.float32),
    )(cnt16, agg, h, Wl, bl.reshape(1, HO), Wr, Whh, bhh.reshape(1, 512))


def _tc_final(cnt16, agg, h, Wl3, bl3, Wr3, Woh, boh, batch2, alpha2, w1a,
              w1b, b11, noise, mask):
    def body(c_ref, a_ref, h_ref, wl_ref, bl_ref, wr_ref, wo_ref, bo_ref,
             bt_ref, al_ref, wa_ref, wb_ref, bb_ref, nz_ref, mk_ref, o_ref):
        inv = 1.0 / jnp.maximum(c_ref[...][:, 0:1], 1.0)
        mean = a_ref[...] * inv
        t = (jnp.dot(mean, wl_ref[...], preferred_element_type=jnp.float32)
             + bl_ref[...]
             + jnp.dot(h_ref[...], wr_ref[...],
                       preferred_element_type=jnp.float32))
        t = jnp.maximum(t, 0.0)
        hf = jnp.dot(t, wo_ref[...], preferred_element_type=jnp.float32) \
            + bo_ref[...]
        hf = jnp.where(hf >= 0, hf, 0.01 * hf)
        bi = lax.broadcasted_iota(jnp.int32, (_R, 64), 1)
        oh = (bt_ref[...] == bi).astype(jnp.float32)
        an = jnp.dot(oh, al_ref[...], preferred_element_type=jnp.float32)
        z = (jnp.dot(hf, wa_ref[...], preferred_element_type=jnp.float32)
             + an * wb_ref[...] + bb_ref[...] + nz_ref[...])
        o_ref[...] = 10.0 * jax.nn.sigmoid(z) * mk_ref[...]

    return pl.pallas_call(
        body,
        grid=(N // _R,),
        in_specs=[pl.BlockSpec((_R, 128), lambda i: (i, 0)),
                  pl.BlockSpec((_R, 512), lambda i: (i, 0)),
                  pl.BlockSpec((_R, 512), lambda i: (i, 0)),
                  pl.BlockSpec((512, 256), lambda i: (0, 0)),
                  pl.BlockSpec((1, 256), lambda i: (0, 0)),
                  pl.BlockSpec((512, 256), lambda i: (0, 0)),
                  pl.BlockSpec((256, 512), lambda i: (0, 0)),
                  pl.BlockSpec((1, 512), lambda i: (0, 0)),
                  pl.BlockSpec((_R, 1), lambda i: (i, 0)),
                  pl.BlockSpec((64, 1), lambda i: (0, 0)),
                  pl.BlockSpec((512, 1), lambda i: (0, 0)),
                  pl.BlockSpec((1, 1), lambda i: (0, 0)),
                  pl.BlockSpec((1, 1), lambda i: (0, 0)),
                  pl.BlockSpec((_R, 1), lambda i: (i, 0)),
                  pl.BlockSpec((_R, 1), lambda i: (i, 0))],
        out_specs=pl.BlockSpec((_R, 1), lambda i: (i, 0)),
        out_shape=jax.ShapeDtypeStruct((N, 1), jnp.float32),
    )(cnt16, agg, h, Wl3, bl3.reshape(1, 256), Wr3, Woh, boh.reshape(1, 512),
      batch2, alpha2, w1a, w1b, b11, noise, mask)


def kernel(x, alpha, edge_index, batch, W_pre, b_pre, Wl1, bl1, Wr1, Wl2, bl2,
           Wr2, Wl3, bl3, Wr3, W_hh1, b_hh1, W_hh2, b_hh2, W_oh, b_oh, W_h1,
           b_h1):
    src, dst = edge_index[0], edge_index[1]
    fixed_feature = x[:, :10]
    mask = x[:, 9:10]

    u = jax.random.uniform(jax.random.key(12345), (N, 1), minval=1e-10,
                           maxval=1.0 - 1e-10, dtype=jnp.float32)
    noise = jnp.log(u) - jnp.log(1.0 - u)

    npad_e = EPP - EP
    padsrc = jnp.arange(npad_e, dtype=jnp.int32).reshape(1, npad_e)
    paddst = (N + jnp.arange(npad_e, dtype=jnp.int32)).reshape(1, npad_e)
    srcp = jnp.concatenate(
        [src.reshape(NT, EP), jnp.broadcast_to(padsrc, (NT, npad_e))], axis=1)
    dstp = jnp.concatenate(
        [dst.reshape(NT, EP), jnp.broadcast_to(paddst, (NT, npad_e))], axis=1)

    h0 = _tc_pre(x, W_pre, b_pre)
    cnt128 = _sc_counts(dstp)
    agg1 = _seg_sum(h0, srcp, dstp)
    h1 = _tc_layer(cnt128, agg1, h0, Wl1, bl1, Wr1, W_hh1, b_hh1)
    agg2 = _seg_sum(h1, srcp, dstp)
    h2 = _tc_layer(cnt128, agg2, h1, Wl2, bl2, Wr2, W_hh2, b_hh2)
    agg3 = _seg_sum(h2, srcp, dstp)
    out = _tc_final(cnt128, agg3, h2, Wl3, bl3, Wr3, W_oh, b_oh,
                    batch.reshape(N, 1), alpha.reshape(64, 1),
                    W_h1[:512], W_h1[512:513], b_h1.reshape(1, 1),
                    noise, mask)
    return (out, fixed_feature)


# speedup vs baseline: 4.8230x; 1.2325x over previous
import functools

import jax
import jax.numpy as jnp
from jax import lax
from jax.experimental import pallas as pl
from jax.experimental.pallas import tpu as pltpu
from jax.experimental.pallas import tpu_sc as plsc

N = 10000
E = 160000
NCHUNK = 4
CW = 128
NT = 16
EP = E // NT
KB = 128
NB = 80
EPP = NB * KB
NPAD = 10240
RPT = NPAD // NT
G = 8
NGRP = NB // G


def _seg_sum_body(h, srcp4, dstp4, zhbm, out, idx2d, dst2d,
                  gbuf_a, gbuf_b, acc, sem_a, sem_b):
    s = lax.axis_index("s")
    c = lax.axis_index("c")

    for j in range(RPT // KB):
        pltpu.sync_copy(zhbm, acc.at[pl.ds(s * RPT + j * KB, KB)])
    plsc.subcore_barrier()

    for cabs in range(NCHUNK):
        own = c == cabs // (NCHUNK // 2)
        cs = cabs * CW

        @pl.when(own)
        def _scan(cabs=cabs, cs=cs):
            def group_body(g, carry):
                pltpu.sync_copy(srcp4.at[s, g], idx2d)
                pltpu.sync_copy(dstp4.at[s, g], dst2d)
                pltpu.async_copy(h.at[idx2d.at[0], pl.ds(cs, CW)],
                                 gbuf_a, sem_a)

                def pair_body(k, carry2):
                    j0 = 2 * k
                    j1 = 2 * k + 1
                    pltpu.async_copy(h.at[idx2d.at[j1], pl.ds(cs, CW)],
                                     gbuf_b, sem_b)
                    pltpu.make_async_copy(h.at[idx2d.at[j0], pl.ds(cs, CW)],
                                          gbuf_a, sem_a).wait()
                    pltpu.sync_copy(gbuf_a, acc.at[dst2d.at[j0]], add=True)

                    @pl.when(k < G // 2 - 1)
                    def _prefetch():
                        pltpu.async_copy(
                            h.at[idx2d.at[j0 + 2], pl.ds(cs, CW)],
                            gbuf_a, sem_a)

                    pltpu.make_async_copy(h.at[idx2d.at[j1], pl.ds(cs, CW)],
                                          gbuf_b, sem_b).wait()
                    pltpu.sync_copy(gbuf_b, acc.at[dst2d.at[j1]], add=True)
                    return carry2

                lax.fori_loop(0, G // 2, pair_body, 0)
                return carry

            lax.fori_loop(0, NGRP, group_body, 0)

        plsc.subcore_barrier()

        @pl.when(own)
        def _flush(cabs=cabs, cs=cs):
            for j in range(RPT // KB):
                r0 = s * RPT + j * KB
                pltpu.sync_copy(acc.at[pl.ds(r0, KB)],
                                out.at[pl.ds(r0, KB), pl.ds(cs, CW)])
                pltpu.sync_copy(zhbm, acc.at[pl.ds(r0, KB)])

        plsc.subcore_barrier()


def _seg_sum(h, srcp, dstp):
    zhbm = jnp.zeros((KB, CW), jnp.float32)
    mesh = plsc.VectorSubcoreMesh(core_axis_name="c", subcore_axis_name="s")
    return pl.kernel(
        _seg_sum_body,
        out_type=jax.ShapeDtypeStruct((NPAD, 512), jnp.float32),
        mesh=mesh,
        scratch_types=[
            pltpu.VMEM((G, KB), jnp.int32),
            pltpu.VMEM((G, KB), jnp.int32),
            pltpu.VMEM((KB, CW), jnp.float32),
            pltpu.VMEM((KB, CW), jnp.float32),
            pltpu.VMEM_SHARED((NPAD, CW), jnp.float32),
            pltpu.SemaphoreType.DMA,
            pltpu.SemaphoreType.DMA,
        ],
    )(h, srcp.reshape(NT, NGRP, G, KB), dstp.reshape(NT, NGRP, G, KB), zhbm)


def _cnt_body(dstp, onesb, zhbm, out, dst_cur, onesv, acc, sem):
    s = lax.axis_index("s")
    c = lax.axis_index("c")

    @pl.when(c == 0)
    def _zero():
        for j in range(RPT // KB):
            pltpu.sync_copy(zhbm, acc.at[pl.ds(s * RPT + j * KB, KB)])
        pltpu.sync_copy(onesb, onesv)

    plsc.subcore_barrier()

    @pl.when(c == 0)
    def _scan():
        def batch_body(b, carry):
            pltpu.sync_copy(dstp.at[s, pl.ds(b * KB, KB)], dst_cur)
            pltpu.sync_copy(onesv, acc.at[dst_cur], add=True)
            return carry

        lax.fori_loop(0, NB, batch_body, 0)

    plsc.subcore_barrier()

    @pl.when(c == 0)
    def _flush():
        for j in range(RPT // KB):
            r0 = s * RPT + j * KB
            pltpu.sync_copy(acc.at[pl.ds(r0, KB)], out.at[pl.ds(r0, KB)])


def _sc_counts(dstp):
    onesb = jnp.ones((KB, CW), jnp.float32)
    zhbm = jnp.zeros((KB, CW), jnp.float32)
    mesh = plsc.VectorSubcoreMesh(core_axis_name="c", subcore_axis_name="s")
    return pl.kernel(
        _cnt_body,
        out_type=jax.ShapeDtypeStruct((NPAD, CW), jnp.float32),
        mesh=mesh,
        scratch_types=[
            pltpu.VMEM((KB,), jnp.int32),
            pltpu.VMEM((KB, CW), jnp.float32),
            pltpu.VMEM_SHARED((NPAD, CW), jnp.float32),
            pltpu.SemaphoreType.DMA,
        ],
    )(dstp, onesb, zhbm)


_R = 1000


def _tc_pre(x, Wp, bp):
    def body(x_ref, w_ref, b_ref, o_ref):
        o_ref[...] = jnp.dot(x_ref[...], w_ref[...],
                             preferred_element_type=jnp.float32) + b_ref[...]

    return pl.pallas_call(
        body,
        grid=(N // _R,),
        in_specs=[pl.BlockSpec((_R, 256), lambda i: (i, 0)),
                  pl.BlockSpec((256, 512), lambda i: (0, 0)),
                  pl.BlockSpec((1, 512), lambda i: (0, 0))],
        out_specs=pl.BlockSpec((_R, 512), lambda i: (i, 0)),
        out_shape=jax.ShapeDtypeStruct((N, 512), jnp.float32),
    )(x, Wp, bp.reshape(1, 512))


def _tc_layer(cnt16, agg, h, Wl, bl, Wr, Whh, bhh):
    HO = Wl.shape[1]

    def body(c_ref, a_ref, h_ref, wl_ref, bl_ref, wr_ref, wh_ref, bh_ref,
             o_ref):
        inv = 1.0 / jnp.maximum(c_ref[...][:, 0:1], 1.0)
        mean = a_ref[...] * inv
        t = (jnp.dot(mean, wl_ref[...], preferred_element_type=jnp.float32)
             + bl_ref[...]
             + jnp.dot(h_ref[...], wr_ref[...],
                       preferred_element_type=jnp.float32))
        t = jnp.maximum(t, 0.0)
        u = jnp.dot(t, wh_ref[...], preferred_element_type=jnp.float32) \
            + bh_ref[...]
        o_ref[...] = jnp.where(u >= 0, u, 0.01 * u)

    return pl.pallas_call(
        body,
        grid=(N // _R,),
        in_specs=[pl.BlockSpec((_R, 128), lambda i: (i, 0)),
                  pl.BlockSpec((_R, 512), lambda i: (i, 0)),
                  pl.BlockSpec((_R, 512), lambda i: (i, 0)),
                  pl.BlockSpec((512, HO), lambda i: (0, 0)),
                  pl.BlockSpec((1, HO), lambda i: (0, 0)),
                  pl.BlockSpec((512, HO), lambda i: (0, 0)),
                  pl.BlockSpec((HO, 512), lambda i: (0, 0)),
                  pl.BlockSpec((1, 512), lambda i: (0, 0))],
        out_specs=pl.BlockSpec((_R, 512), lambda i: (i, 0)),
        out_shape=jax.ShapeDtypeStruct((N, 512), jnp.float32),
    )(cnt16, agg, h, Wl, bl.reshape(1, HO), Wr, Whh, bhh.reshape(1, 512))


def _tc_final(cnt16, agg, h, Wl3, bl3, Wr3, Woh, boh, batch2, alpha2, w1a,
              w1b, b11, noise, mask):
    def body(c_ref, a_ref, h_ref, wl_ref, bl_ref, wr_ref, wo_ref, bo_ref,
             bt_ref, al_ref, wa_ref, wb_ref, bb_ref, nz_ref, mk_ref, o_ref):
        inv = 1.0 / jnp.maximum(c_ref[...][:, 0:1], 1.0)
        mean = a_ref[...] * inv
        t = (jnp.dot(mean, wl_ref[...], preferred_element_type=jnp.float32)
             + bl_ref[...]
             + jnp.dot(h_ref[...], wr_ref[...],
                       preferred_element_type=jnp.float32))
        t = jnp.maximum(t, 0.0)
        hf = jnp.dot(t, wo_ref[...], preferred_element_type=jnp.float32) \
            + bo_ref[...]
        hf = jnp.where(hf >= 0, hf, 0.01 * hf)
        bi = lax.broadcasted_iota(jnp.int32, (_R, 64), 1)
        oh = (bt_ref[...] == bi).astype(jnp.float32)
        an = jnp.dot(oh, al_ref[...], preferred_element_type=jnp.float32)
        z = (jnp.dot(hf, wa_ref[...], preferred_element_type=jnp.float32)
             + an * wb_ref[...] + bb_ref[...] + nz_ref[...])
        o_ref[...] = 10.0 * jax.nn.sigmoid(z) * mk_ref[...]

    return pl.pallas_call(
        body,
        grid=(N // _R,),
        in_specs=[pl.BlockSpec((_R, 128), lambda i: (i, 0)),
                  pl.BlockSpec((_R, 512), lambda i: (i, 0)),
                  pl.BlockSpec((_R, 512), lambda i: (i, 0)),
                  pl.BlockSpec((512, 256), lambda i: (0, 0)),
                  pl.BlockSpec((1, 256), lambda i: (0, 0)),
                  pl.BlockSpec((512, 256), lambda i: (0, 0)),
                  pl.BlockSpec((256, 512), lambda i: (0, 0)),
                  pl.BlockSpec((1, 512), lambda i: (0, 0)),
                  pl.BlockSpec((_R, 1), lambda i: (i, 0)),
                  pl.BlockSpec((64, 1), lambda i: (0, 0)),
                  pl.BlockSpec((512, 1), lambda i: (0, 0)),
                  pl.BlockSpec((1, 1), lambda i: (0, 0)),
                  pl.BlockSpec((1, 1), lambda i: (0, 0)),
                  pl.BlockSpec((_R, 1), lambda i: (i, 0)),
                  pl.BlockSpec((_R, 1), lambda i: (i, 0))],
        out_specs=pl.BlockSpec((_R, 1), lambda i: (i, 0)),
        out_shape=jax.ShapeDtypeStruct((N, 1), jnp.float32),
    )(cnt16, agg, h, Wl3, bl3.reshape(1, 256), Wr3, Woh, boh.reshape(1, 512),
      batch2, alpha2, w1a, w1b, b11, noise, mask)


def kernel(x, alpha, edge_index, batch, W_pre, b_pre, Wl1, bl1, Wr1, Wl2, bl2,
           Wr2, Wl3, bl3, Wr3, W_hh1, b_hh1, W_hh2, b_hh2, W_oh, b_oh, W_h1,
           b_h1):
    src, dst = edge_index[0], edge_index[1]
    fixed_feature = x[:, :10]
    mask = x[:, 9:10]

    u = jax.random.uniform(jax.random.key(12345), (N, 1), minval=1e-10,
                           maxval=1.0 - 1e-10, dtype=jnp.float32)
    noise = jnp.log(u) - jnp.log(1.0 - u)

    npad_e = EPP - EP
    padsrc = jnp.arange(npad_e, dtype=jnp.int32).reshape(1, npad_e)
    paddst = (N + jnp.arange(npad_e, dtype=jnp.int32)).reshape(1, npad_e)
    srcp = jnp.concatenate(
        [src.reshape(NT, EP), jnp.broadcast_to(padsrc, (NT, npad_e))], axis=1)
    dstp = jnp.concatenate(
        [dst.reshape(NT, EP), jnp.broadcast_to(paddst, (NT, npad_e))], axis=1)

    h0 = _tc_pre(x, W_pre, b_pre)
    cnt128 = _sc_counts(dstp)
    agg1 = _seg_sum(h0, srcp, dstp)
    h1 = _tc_layer(cnt128, agg1, h0, Wl1, bl1, Wr1, W_hh1, b_hh1)
    agg2 = _seg_sum(h1, srcp, dstp)
    h2 = _tc_layer(cnt128, agg2, h1, Wl2, bl2, Wr2, W_hh2, b_hh2)
    agg3 = _seg_sum(h2, srcp, dstp)
    out = _tc_final(cnt128, agg3, h2, Wl3, bl3, Wr3, W_oh, b_oh,
                    batch.reshape(N, 1), alpha.reshape(64, 1),
                    W_h1[:512], W_h1[512:513], b_h1.reshape(1, 1),
                    noise, mask)
    return (out, fixed_feature)


# speedup vs baseline: 5.2998x; 1.0989x over previous
import functools

import jax
import jax.numpy as jnp
from jax import lax
from jax.experimental import pallas as pl
from jax.experimental.pallas import tpu as pltpu
from jax.experimental.pallas import tpu_sc as plsc

N = 10000
E = 160000
NCHUNK = 4
CW = 128
NT = 16
EP = E // NT
KB = 128
NB = 80
EPP = NB * KB
NPAD = 10240
RPT = NPAD // NT
G = 8
NGRP = NB // G


def _seg_sum_body(h, srcp4, dstp4, zhbm, out, idx2d_a, idx2d_b, dst2d_a,
                  dst2d_b, gbuf_a, gbuf_b, acc, sem_a, sem_b, sem_ia,
                  sem_ib):
    s = lax.axis_index("s")
    c = lax.axis_index("c")

    for j in range(RPT // KB):
        pltpu.sync_copy(zhbm, acc.at[pl.ds(s * RPT + j * KB, KB)])
    plsc.subcore_barrier()

    for cabs in range(NCHUNK):
        own = c == cabs // (NCHUNK // 2)
        cs = cabs * CW

        @pl.when(own)
        def _scan(cabs=cabs, cs=cs):
            def process_group(idxb, dstb):
                pltpu.async_copy(h.at[idxb.at[0], pl.ds(cs, CW)],
                                 gbuf_a, sem_a)

                def pair_body(k, carry2):
                    j0 = 2 * k
                    j1 = 2 * k + 1
                    pltpu.async_copy(h.at[idxb.at[j1], pl.ds(cs, CW)],
                                     gbuf_b, sem_b)
                    pltpu.make_async_copy(h.at[idxb.at[j0], pl.ds(cs, CW)],
                                          gbuf_a, sem_a).wait()
                    pltpu.sync_copy(gbuf_a, acc.at[dstb.at[j0]], add=True)

                    @pl.when(k < G // 2 - 1)
                    def _prefetch():
                        pltpu.async_copy(
                            h.at[idxb.at[j0 + 2], pl.ds(cs, CW)],
                            gbuf_a, sem_a)

                    pltpu.make_async_copy(h.at[idxb.at[j1], pl.ds(cs, CW)],
                                          gbuf_b, sem_b).wait()
                    pltpu.sync_copy(gbuf_b, acc.at[dstb.at[j1]], add=True)
                    return carry2

                lax.fori_loop(0, G // 2, pair_body, 0)

            pltpu.async_copy(srcp4.at[s, 0], idx2d_a, sem_ia)
            pltpu.async_copy(dstp4.at[s, 0], dst2d_a, sem_ia)

            def pg_body(p, carry):
                g1 = 2 * p + 1
                pltpu.async_copy(srcp4.at[s, g1], idx2d_b, sem_ib)
                pltpu.async_copy(dstp4.at[s, g1], dst2d_b, sem_ib)
                pltpu.make_async_copy(srcp4.at[s, g1], idx2d_a, sem_ia).wait()
                pltpu.make_async_copy(dstp4.at[s, g1], dst2d_a, sem_ia).wait()
                process_group(idx2d_a, dst2d_a)

                @pl.when(p < NGRP // 2 - 1)
                def _stage_next():
                    g2 = 2 * p + 2
                    pltpu.async_copy(srcp4.at[s, g2], idx2d_a, sem_ia)
                    pltpu.async_copy(dstp4.at[s, g2], dst2d_a, sem_ia)

                pltpu.make_async_copy(srcp4.at[s, g1], idx2d_b, sem_ib).wait()
                pltpu.make_async_copy(dstp4.at[s, g1], dst2d_b, sem_ib).wait()
                process_group(idx2d_b, dst2d_b)
                return carry

            lax.fori_loop(0, NGRP // 2, pg_body, 0)

        plsc.subcore_barrier()

        @pl.when(own)
        def _flush(cabs=cabs, cs=cs):
            for j in range(RPT // KB):
                r0 = s * RPT + j * KB
                pltpu.sync_copy(acc.at[pl.ds(r0, KB)],
                                out.at[pl.ds(r0, KB), pl.ds(cs, CW)])
                pltpu.sync_copy(zhbm, acc.at[pl.ds(r0, KB)])

        plsc.subcore_barrier()


def _seg_sum(h, srcp, dstp):
    zhbm = jnp.zeros((KB, CW), jnp.float32)
    mesh = plsc.VectorSubcoreMesh(core_axis_name="c", subcore_axis_name="s")
    return pl.kernel(
        _seg_sum_body,
        out_type=jax.ShapeDtypeStruct((NPAD, 512), jnp.float32),
        mesh=mesh,
        scratch_types=[
            pltpu.VMEM((G, KB), jnp.int32),
            pltpu.VMEM((G, KB), jnp.int32),
            pltpu.VMEM((G, KB), jnp.int32),
            pltpu.VMEM((G, KB), jnp.int32),
            pltpu.VMEM((KB, CW), jnp.float32),
            pltpu.VMEM((KB, CW), jnp.float32),
            pltpu.VMEM_SHARED((NPAD, CW), jnp.float32),
            pltpu.SemaphoreType.DMA,
            pltpu.SemaphoreType.DMA,
            pltpu.SemaphoreType.DMA,
            pltpu.SemaphoreType.DMA,
        ],
    )(h, srcp.reshape(NT, NGRP, G, KB), dstp.reshape(NT, NGRP, G, KB), zhbm)


EP32 = E // 32
NB32 = 40
EPP32 = NB32 * KB
NGRP32 = NB32 // G


def _cnt_body(dstp4, onesb, zhbm, out0, out1, dst2d, onesv, acc, sem):
    s = lax.axis_index("s")
    c = lax.axis_index("c")
    w = c * NT + s

    for j in range(RPT // KB):
        pltpu.sync_copy(zhbm, acc.at[pl.ds(s * RPT + j * KB, KB)])
    pltpu.sync_copy(onesb, onesv)
    plsc.subcore_barrier()

    def group_body(g, carry):
        pltpu.sync_copy(dstp4.at[w, g], dst2d)

        def batch_body(k, carry2):
            pltpu.sync_copy(onesv, acc.at[dst2d.at[k]], add=True)
            return carry2

        lax.fori_loop(0, G, batch_body, 0)
        return carry

    lax.fori_loop(0, NGRP32, group_body, 0)
    plsc.subcore_barrier()

    for j in range(RPT // KB):
        r0 = s * RPT + j * KB

        @pl.when(c == 0)
        def _f0(r0=r0):
            pltpu.sync_copy(acc.at[pl.ds(r0, KB)], out0.at[pl.ds(r0, KB)])

        @pl.when(c == 1)
        def _f1(r0=r0):
            pltpu.sync_copy(acc.at[pl.ds(r0, KB)], out1.at[pl.ds(r0, KB)])


def _sc_counts(dstp32):
    onesb = jnp.ones((KB, CW), jnp.float32)
    zhbm = jnp.zeros((KB, CW), jnp.float32)
    mesh = plsc.VectorSubcoreMesh(core_axis_name="c", subcore_axis_name="s")
    return pl.kernel(
        _cnt_body,
        out_type=(jax.ShapeDtypeStruct((NPAD, CW), jnp.float32),
                  jax.ShapeDtypeStruct((NPAD, CW), jnp.float32)),
        mesh=mesh,
        scratch_types=[
            pltpu.VMEM((G, KB), jnp.int32),
            pltpu.VMEM((KB, CW), jnp.float32),
            pltpu.VMEM_SHARED((NPAD, CW), jnp.float32),
            pltpu.SemaphoreType.DMA,
        ],
    )(dstp32.reshape(32, NGRP32, G, KB), onesb, zhbm)


_R = 1000


def _tc_pre(x, Wp, bp):
    def body(x_ref, w_ref, b_ref, o_ref):
        o_ref[...] = jnp.dot(x_ref[...], w_ref[...],
                             preferred_element_type=jnp.float32) + b_ref[...]

    return pl.pallas_call(
        body,
        grid=(N // _R,),
        in_specs=[pl.BlockSpec((_R, 256), lambda i: (i, 0)),
                  pl.BlockSpec((256, 512), lambda i: (0, 0)),
                  pl.BlockSpec((1, 512), lambda i: (0, 0))],
        out_specs=pl.BlockSpec((_R, 512), lambda i: (i, 0)),
        out_shape=jax.ShapeDtypeStruct((N, 512), jnp.float32),
    )(x, Wp, bp.reshape(1, 512))


def _tc_layer(cnta, cntb, agg, h, Wl, bl, Wr, Whh, bhh):
    HO = Wl.shape[1]

    def body(ca_ref, cb_ref, a_ref, h_ref, wl_ref, bl_ref, wr_ref, wh_ref,
             bh_ref, o_ref):
        inv = 1.0 / jnp.maximum(ca_ref[...][:, 0:1] + cb_ref[...][:, 0:1],
                                1.0)
        mean = a_ref[...] * inv
        t = (jnp.dot(mean, wl_ref[...], preferred_element_type=jnp.float32)
             + bl_ref[...]
             + jnp.dot(h_ref[...], wr_ref[...],
                       preferred_element_type=jnp.float32))
        t = jnp.maximum(t, 0.0)
        u = jnp.dot(t, wh_ref[...], preferred_element_type=jnp.float32) \
            + bh_ref[...]
        o_ref[...] = jnp.where(u >= 0, u, 0.01 * u)

    return pl.pallas_call(
        body,
        grid=(N // _R,),
        in_specs=[pl.BlockSpec((_R, 128), lambda i: (i, 0)),
                  pl.BlockSpec((_R, 128), lambda i: (i, 0)),
                  pl.BlockSpec((_R, 512), lambda i: (i, 0)),
                  pl.BlockSpec((_R, 512), lambda i: (i, 0)),
                  pl.BlockSpec((512, HO), lambda i: (0, 0)),
                  pl.BlockSpec((1, HO), lambda i: (0, 0)),
                  pl.BlockSpec((512, HO), lambda i: (0, 0)),
                  pl.BlockSpec((HO, 512), lambda i: (0, 0)),
                  pl.BlockSpec((1, 512), lambda i: (0, 0))],
        out_specs=pl.BlockSpec((_R, 512), lambda i: (i, 0)),
        out_shape=jax.ShapeDtypeStruct((N, 512), jnp.float32),
    )(cnta, cntb, agg, h, Wl, bl.reshape(1, HO), Wr, Whh,
      bhh.reshape(1, 512))


def _tc_final(cnta, cntb, agg, h, Wl3, bl3, Wr3, Woh, boh, batch2, alpha2,
              w1a, w1b, b11, noise, mask):
    def body(ca_ref, cb_ref, a_ref, h_ref, wl_ref, bl_ref, wr_ref, wo_ref,
             bo_ref, bt_ref, al_ref, wa_ref, wb_ref, bb_ref, nz_ref, mk_ref,
             o_ref):
        inv = 1.0 / jnp.maximum(ca_ref[...][:, 0:1] + cb_ref[...][:, 0:1],
                                1.0)
        mean = a_ref[...] * inv
        t = (jnp.dot(mean, wl_ref[...], preferred_element_type=jnp.float32)
             + bl_ref[...]
             + jnp.dot(h_ref[...], wr_ref[...],
                       preferred_element_type=jnp.float32))
        t = jnp.maximum(t, 0.0)
        hf = jnp.dot(t, wo_ref[...], preferred_element_type=jnp.float32) \
            + bo_ref[...]
        hf = jnp.where(hf >= 0, hf, 0.01 * hf)
        bi = lax.broadcasted_iota(jnp.int32, (_R, 64), 1)
        oh = (bt_ref[...] == bi).astype(jnp.float32)
        an = jnp.dot(oh, al_ref[...], preferred_element_type=jnp.float32)
        z = (jnp.dot(hf, wa_ref[...], preferred_element_type=jnp.float32)
             + an * wb_ref[...] + bb_ref[...] + nz_ref[...])
        o_ref[...] = 10.0 * jax.nn.sigmoid(z) * mk_ref[...]

    return pl.pallas_call(
        body,
        grid=(N // _R,),
        in_specs=[pl.BlockSpec((_R, 128), lambda i: (i, 0)),
                  pl.BlockSpec((_R, 128), lambda i: (i, 0)),
                  pl.BlockSpec((_R, 512), lambda i: (i, 0)),
                  pl.BlockSpec((_R, 512), lambda i: (i, 0)),
                  pl.BlockSpec((512, 256), lambda i: (0, 0)),
                  pl.BlockSpec((1, 256), lambda i: (0, 0)),
                  pl.BlockSpec((512, 256), lambda i: (0, 0)),
                  pl.BlockSpec((256, 512), lambda i: (0, 0)),
                  pl.BlockSpec((1, 512), lambda i: (0, 0)),
                  pl.BlockSpec((_R, 1), lambda i: (i, 0)),
                  pl.BlockSpec((64, 1), lambda i: (0, 0)),
                  pl.BlockSpec((512, 1), lambda i: (0, 0)),
                  pl.BlockSpec((1, 1), lambda i: (0, 0)),
                  pl.BlockSpec((1, 1), lambda i: (0, 0)),
                  pl.BlockSpec((_R, 1), lambda i: (i, 0)),
                  pl.BlockSpec((_R, 1), lambda i: (i, 0))],
        out_specs=pl.BlockSpec((_R, 1), lambda i: (i, 0)),
        out_shape=jax.ShapeDtypeStruct((N, 1), jnp.float32),
    )(cnta, cntb, agg, h, Wl3, bl3.reshape(1, 256), Wr3, Woh,
      boh.reshape(1, 512), batch2, alpha2, w1a, w1b, b11, noise, mask)


def kernel(x, alpha, edge_index, batch, W_pre, b_pre, Wl1, bl1, Wr1, Wl2, bl2,
           Wr2, Wl3, bl3, Wr3, W_hh1, b_hh1, W_hh2, b_hh2, W_oh, b_oh, W_h1,
           b_h1):
    src, dst = edge_index[0], edge_index[1]
    fixed_feature = x[:, :10]
    mask = x[:, 9:10]

    u = jax.random.uniform(jax.random.key(12345), (N, 1), minval=1e-10,
                           maxval=1.0 - 1e-10, dtype=jnp.float32)
    noise = jnp.log(u) - jnp.log(1.0 - u)

    npad_e = EPP - EP
    padsrc = jnp.arange(npad_e, dtype=jnp.int32).reshape(1, npad_e)
    paddst = (N + jnp.arange(npad_e, dtype=jnp.int32)).reshape(1, npad_e)
    srcp = jnp.concatenate(
        [src.reshape(NT, EP), jnp.broadcast_to(padsrc, (NT, npad_e))], axis=1)
    dstp = jnp.concatenate(
        [dst.reshape(NT, EP), jnp.broadcast_to(paddst, (NT, npad_e))], axis=1)

    npad32 = EPP32 - EP32
    paddst32 = (N + jnp.arange(npad32, dtype=jnp.int32)).reshape(1, npad32)
    dstp32 = jnp.concatenate(
        [dst.reshape(32, EP32), jnp.broadcast_to(paddst32, (32, npad32))],
        axis=1)

    h0 = _tc_pre(x, W_pre, b_pre)
    cnt_a, cnt_b = _sc_counts(dstp32)
    agg1 = _seg_sum(h0, srcp, dstp)
    h1 = _tc_layer(cnt_a, cnt_b, agg1, h0, Wl1, bl1, Wr1, W_hh1, b_hh1)
    agg2 = _seg_sum(h1, srcp, dstp)
    h2 = _tc_layer(cnt_a, cnt_b, agg2, h1, Wl2, bl2, Wr2, W_hh2, b_hh2)
    agg3 = _seg_sum(h2, srcp, dstp)
    out = _tc_final(cnt_a, cnt_b, agg3, h2, Wl3, bl3, Wr3, W_oh, b_oh,
                    batch.reshape(N, 1), alpha.reshape(64, 1),
                    W_h1[:512], W_h1[512:513], b_h1.reshape(1, 1),
                    noise, mask)
    return (out, fixed_feature)
